# baseline (speedup 1.0000x reference)
import functools
import jax
import jax.numpy as jnp
from jax import lax
from jax.experimental import pallas as pl
from jax.experimental.pallas import tpu as pltpu
from jax.experimental.pallas import tpu_sc as plsc

N_NODES = 10000
P_NODES = 10240
DIM = 128
N_EDGES = 320000
NCORES = 2
NSUB = 16
NW = NCORES * NSUB
CHUNK = 128
CHUNKS = 80
E_PAD = NW * CHUNKS * CHUNK
ROWS_PER_TILE = P_NODES // NSUB

_mesh = plsc.VectorSubcoreMesh(core_axis_name="c", subcore_axis_name="s")


def _zero16():
    return jnp.zeros((16,), jnp.float32)


@functools.partial(
    pl.kernel,
    mesh=_mesh,
    out_type=jax.ShapeDtypeStruct((NCORES, P_NODES, DIM), jnp.float32),
    scratch_types=[
        pltpu.VMEM((CHUNKS, CHUNK), jnp.int32),
        pltpu.VMEM((CHUNK, DIM), jnp.float32),
        pltpu.VMEM_SHARED((P_NODES, DIM), jnp.float32),
        pltpu.SemaphoreType.DMA,
    ],
)
def _sc_deg(dst_hbm, out_hbm, dst_all, rows_v, acc, sem):
    c = lax.axis_index("c")
    s = lax.axis_index("s")
    wid = c * NSUB + s
    pltpu.sync_copy(dst_hbm.at[wid], dst_all)

    zero16 = _zero16()
    one16 = jnp.full((16,), 1.0, jnp.float32)

    def zero_body(i, carry):
        for j in range(DIM // 16):
            rows_v[i, pl.ds(j * 16, 16)] = zero16
        return carry

    lax.fori_loop(0, CHUNK, zero_body, 0)

    for k in range(ROWS_PER_TILE // CHUNK):
        pltpu.sync_copy(rows_v, acc.at[pl.ds(s * ROWS_PER_TILE + k * CHUNK, CHUNK)])

    def ones_body(i, carry):
        for j in range(DIM // 16):
            rows_v[i, pl.ds(j * 16, 16)] = one16
        return carry

    lax.fori_loop(0, CHUNK, ones_body, 0)
    plsc.subcore_barrier()

    def loop_body(ci, carry):
        pltpu.sync_copy(rows_v, acc.at[dst_all.at[ci]], add=True)
        return carry

    lax.fori_loop(0, CHUNKS, loop_body, 0)
    plsc.subcore_barrier()

    pltpu.sync_copy(
        acc.at[pl.ds(s * ROWS_PER_TILE, ROWS_PER_TILE)],
        out_hbm.at[c, pl.ds(s * ROWS_PER_TILE, ROWS_PER_TILE)],
    )


@functools.partial(
    pl.kernel,
    mesh=_mesh,
    out_type=jax.ShapeDtypeStruct((NCORES, P_NODES, DIM), jnp.float32),
    scratch_types=[
        pltpu.VMEM((CHUNK, DIM), jnp.float32),
        pltpu.VMEM((CHUNK, DIM), jnp.float32),
        pltpu.VMEM((2, CHUNK), jnp.int32),
        pltpu.VMEM((2, CHUNK), jnp.int32),
        pltpu.VMEM_SHARED((P_NODES, DIM), jnp.float32),
        pltpu.SemaphoreType.DMA,
        pltpu.SemaphoreType.DMA,
    ],
)
def _sc_scatter(t_hbm, e_hbm, out_hbm, b0, b1, i0, i1, acc, g0, g1):
    c = lax.axis_index("c")
    s = lax.axis_index("s")
    wid = c * NSUB + s
    bufs = [b0, b1]
    ibuf = [i0, i1]
    gsem = [g0, g1]

    zero16 = _zero16()

    def zero_body(i, carry):
        for j in range(DIM // 16):
            b0[i, pl.ds(j * 16, 16)] = zero16
        return carry

    lax.fori_loop(0, CHUNK, zero_body, 0)

    for k in range(ROWS_PER_TILE // CHUNK):
        pltpu.sync_copy(b0, acc.at[pl.ds(s * ROWS_PER_TILE + k * CHUNK, CHUNK)])
    plsc.subcore_barrier()

    def wait_rows(sem, buf):
        pltpu.make_async_copy(t_hbm.at[ibuf[0].at[0]], buf, sem).wait()

    pltpu.sync_copy(e_hbm.at[wid, 0], i0)
    pltpu.async_copy(t_hbm.at[i0.at[0]], b0, g0)

    def step(cur, b):
        o = 1 - b
        wait_rows(gsem[b], bufs[b])

        @pl.when(cur + 1 < CHUNKS)
        def _():
            pltpu.sync_copy(e_hbm.at[wid, cur + 1], ibuf[o])
            pltpu.async_copy(t_hbm.at[ibuf[o].at[0]], bufs[o], gsem[o])

        pltpu.sync_copy(bufs[b], acc.at[ibuf[b].at[1]], add=True)

    def body(i, carry):
        ci = i * 2
        for t in range(2):
            step(ci + t, t)
        return carry

    lax.fori_loop(0, CHUNKS // 2, body, 0)

    plsc.subcore_barrier()
    pltpu.sync_copy(
        acc.at[pl.ds(s * ROWS_PER_TILE, ROWS_PER_TILE)],
        out_hbm.at[c, pl.ds(s * ROWS_PER_TILE, ROWS_PER_TILE)],
    )


BLK = 2000
GRID = N_NODES // BLK


def _tc_first_body(x_ref, w_ref, d0_ref, d1_ref, t_ref, n_ref):
    deg = d0_ref[:, :1] + d1_ref[:, :1]
    norm = jnp.where(deg > 0, lax.rsqrt(deg), 0.0)
    t_ref[...] = jnp.dot(x_ref[...], w_ref[...], preferred_element_type=jnp.float32) * norm
    n_ref[...] = norm


_tc_first = pl.pallas_call(
    _tc_first_body,
    grid=(GRID,),
    in_specs=[
        pl.BlockSpec((BLK, DIM), lambda i: (i, 0)),
        pl.BlockSpec((DIM, DIM), lambda i: (0, 0)),
        pl.BlockSpec((BLK, DIM), lambda i: (i, 0)),
        pl.BlockSpec((BLK, DIM), lambda i: (i, 0)),
    ],
    out_specs=[
        pl.BlockSpec((BLK, DIM), lambda i: (i, 0)),
        pl.BlockSpec((BLK, 1), lambda i: (i, 0)),
    ],
    out_shape=[
        jax.ShapeDtypeStruct((N_NODES, DIM), jnp.float32),
        jax.ShapeDtypeStruct((N_NODES, 1), jnp.float32),
    ],
)


def _tc_mid_body(p0_ref, p1_ref, n_ref, w_ref, t_ref):
    norm = n_ref[...]
    x = jnp.maximum((p0_ref[...] + p1_ref[...]) * norm, 0.0)
    t_ref[...] = jnp.dot(x, w_ref[...], preferred_element_type=jnp.float32) * norm


_tc_mid = pl.pallas_call(
    _tc_mid_body,
    grid=(GRID,),
    in_specs=[
        pl.BlockSpec((BLK, DIM), lambda i: (i, 0)),
        pl.BlockSpec((BLK, DIM), lambda i: (i, 0)),
        pl.BlockSpec((BLK, 1), lambda i: (i, 0)),
        pl.BlockSpec((DIM, DIM), lambda i: (0, 0)),
    ],
    out_specs=pl.BlockSpec((BLK, DIM), lambda i: (i, 0)),
    out_shape=jax.ShapeDtypeStruct((N_NODES, DIM), jnp.float32),
)


def _tc_last_body(p0_ref, p1_ref, n_ref, o_ref):
    o_ref[...] = (p0_ref[...] + p1_ref[...]) * n_ref[...]


_tc_last = pl.pallas_call(
    _tc_last_body,
    grid=(GRID,),
    in_specs=[
        pl.BlockSpec((BLK, DIM), lambda i: (i, 0)),
        pl.BlockSpec((BLK, DIM), lambda i: (i, 0)),
        pl.BlockSpec((BLK, 1), lambda i: (i, 0)),
    ],
    out_specs=pl.BlockSpec((BLK, DIM), lambda i: (i, 0)),
    out_shape=jax.ShapeDtypeStruct((N_NODES, DIM), jnp.float32),
)


def kernel(features, edge_index, W0, W1, W2):
    src = edge_index[0].astype(jnp.int32)
    dst = edge_index[1].astype(jnp.int32)
    n_pad = E_PAD - N_EDGES
    pad_s = jnp.zeros((n_pad,), jnp.int32)
    pad_d = N_NODES + (jnp.arange(n_pad, dtype=jnp.int32) % (P_NODES - N_NODES))
    src_p = jnp.concatenate([src, pad_s]).reshape(NW, CHUNKS, CHUNK)
    dst_p = jnp.concatenate([dst, pad_d]).reshape(NW, CHUNKS, CHUNK)
    edges = jnp.stack([src_p, dst_p], axis=2)

    degp = _sc_deg(dst_p)
    t1, norm = _tc_first(features, W0, degp[0], degp[1])
    p = _sc_scatter(t1, edges)
    t2 = _tc_mid(p[0], p[1], norm, W1)
    p = _sc_scatter(t2, edges)
    t3 = _tc_mid(p[0], p[1], norm, W2)
    p = _sc_scatter(t3, edges)
    return _tc_last(p[0], p[1], norm)


# speedup vs baseline: 1.2833x; 1.2833x over previous
import functools
import jax
import jax.numpy as jnp
from jax import lax
from jax.experimental import pallas as pl
from jax.experimental.pallas import tpu as pltpu
from jax.experimental.pallas import tpu_sc as plsc

N_NODES = 10000
P_NODES = 10240
DIM = 128
N_EDGES = 320000
NCORES = 2
NSUB = 16
NW = NCORES * NSUB
CHUNK = 128
CHUNKS = 79
E_PAD = NW * CHUNKS * CHUNK
ROWS_PER_TILE = P_NODES // NSUB

_mesh = plsc.VectorSubcoreMesh(core_axis_name="c", subcore_axis_name="s")


def _zero16():
    return jnp.zeros((16,), jnp.float32)


@functools.partial(
    pl.kernel,
    mesh=_mesh,
    out_type=jax.ShapeDtypeStruct((NCORES, P_NODES, DIM), jnp.float32),
    scratch_types=[
        pltpu.VMEM((CHUNK,), jnp.int32),
        pltpu.VMEM((CHUNK, DIM), jnp.float32),
        pltpu.VMEM_SHARED((P_NODES, DIM), jnp.float32),
        pltpu.SemaphoreType.DMA,
    ],
)
def _sc_deg(dst_hbm, out_hbm, idx_v, rows_v, acc, sem):
    c = lax.axis_index("c")
    s = lax.axis_index("s")
    wid = c * NSUB + s

    zero16 = _zero16()
    one16 = jnp.full((16,), 1.0, jnp.float32)

    def zero_body(i, carry):
        for j in range(DIM // 16):
            rows_v[i, pl.ds(j * 16, 16)] = zero16
        return carry

    lax.fori_loop(0, CHUNK, zero_body, 0)

    for k in range(ROWS_PER_TILE // CHUNK):
        pltpu.sync_copy(rows_v, acc.at[pl.ds(s * ROWS_PER_TILE + k * CHUNK, CHUNK)])

    def ones_body(i, carry):
        for j in range(DIM // 16):
            rows_v[i, pl.ds(j * 16, 16)] = one16
        return carry

    lax.fori_loop(0, CHUNK, ones_body, 0)
    plsc.subcore_barrier()

    def loop_body(ci, carry):
        pltpu.sync_copy(dst_hbm.at[wid, ci], idx_v)
        pltpu.sync_copy(rows_v, acc.at[idx_v], add=True)
        return carry

    lax.fori_loop(0, CHUNKS, loop_body, 0)
    plsc.subcore_barrier()

    pltpu.sync_copy(
        acc.at[pl.ds(s * ROWS_PER_TILE, ROWS_PER_TILE)],
        out_hbm.at[c, pl.ds(s * ROWS_PER_TILE, ROWS_PER_TILE)],
    )


@functools.partial(
    pl.kernel,
    mesh=_mesh,
    out_type=jax.ShapeDtypeStruct((NCORES, P_NODES, DIM), jnp.float32),
    scratch_types=[
        pltpu.VMEM((CHUNK,), jnp.int32),
        pltpu.VMEM((CHUNK,), jnp.int32),
        pltpu.VMEM((CHUNK, DIM), jnp.float32),
        pltpu.VMEM_SHARED((P_NODES, DIM), jnp.float32),
        pltpu.SemaphoreType.DMA,
    ],
)
def _sc_scatter(t_hbm, src_hbm, dst_hbm, out_hbm, si_v, di_v, rows_v, acc, sem):
    c = lax.axis_index("c")
    s = lax.axis_index("s")
    wid = c * NSUB + s

    zero16 = _zero16()

    def zero_body(i, carry):
        for j in range(DIM // 16):
            rows_v[i, pl.ds(j * 16, 16)] = zero16
        return carry

    lax.fori_loop(0, CHUNK, zero_body, 0)

    for k in range(ROWS_PER_TILE // CHUNK):
        pltpu.sync_copy(rows_v, acc.at[pl.ds(s * ROWS_PER_TILE + k * CHUNK, CHUNK)])
    plsc.subcore_barrier()

    def loop_body(ci, carry):
        pltpu.sync_copy(src_hbm.at[wid, ci], si_v)
        pltpu.sync_copy(dst_hbm.at[wid, ci], di_v)
        pltpu.async_copy(t_hbm.at[si_v], rows_v, sem).wait()
        pltpu.sync_copy(rows_v, acc.at[di_v], add=True)
        return carry

    lax.fori_loop(0, CHUNKS, loop_body, 0)
    plsc.subcore_barrier()

    pltpu.sync_copy(
        acc.at[pl.ds(s * ROWS_PER_TILE, ROWS_PER_TILE)],
        out_hbm.at[c, pl.ds(s * ROWS_PER_TILE, ROWS_PER_TILE)],
    )


BLK = 2000
GRID = N_NODES // BLK


def _tc_first_body(x_ref, w_ref, d0_ref, d1_ref, t_ref, n_ref):
    deg = d0_ref[:, :1] + d1_ref[:, :1]
    norm = jnp.where(deg > 0, lax.rsqrt(deg), 0.0)
    t_ref[...] = jnp.dot(x_ref[...], w_ref[...], preferred_element_type=jnp.float32) * norm
    n_ref[...] = norm


_tc_first = pl.pallas_call(
    _tc_first_body,
    grid=(GRID,),
    in_specs=[
        pl.BlockSpec((BLK, DIM), lambda i: (i, 0)),
        pl.BlockSpec((DIM, DIM), lambda i: (0, 0)),
        pl.BlockSpec((BLK, DIM), lambda i: (i, 0)),
        pl.BlockSpec((BLK, DIM), lambda i: (i, 0)),
    ],
    out_specs=[
        pl.BlockSpec((BLK, DIM), lambda i: (i, 0)),
        pl.BlockSpec((BLK, 1), lambda i: (i, 0)),
    ],
    out_shape=[
        jax.ShapeDtypeStruct((N_NODES, DIM), jnp.float32),
        jax.ShapeDtypeStruct((N_NODES, 1), jnp.float32),
    ],
)


def _tc_mid_body(p0_ref, p1_ref, n_ref, w_ref, t_ref):
    norm = n_ref[...]
    x = jnp.maximum((p0_ref[...] + p1_ref[...]) * norm, 0.0)
    t_ref[...] = jnp.dot(x, w_ref[...], preferred_element_type=jnp.float32) * norm


_tc_mid = pl.pallas_call(
    _tc_mid_body,
    grid=(GRID,),
    in_specs=[
        pl.BlockSpec((BLK, DIM), lambda i: (i, 0)),
        pl.BlockSpec((BLK, DIM), lambda i: (i, 0)),
        pl.BlockSpec((BLK, 1), lambda i: (i, 0)),
        pl.BlockSpec((DIM, DIM), lambda i: (0, 0)),
    ],
    out_specs=pl.BlockSpec((BLK, DIM), lambda i: (i, 0)),
    out_shape=jax.ShapeDtypeStruct((N_NODES, DIM), jnp.float32),
)


def _tc_last_body(p0_ref, p1_ref, n_ref, o_ref):
    o_ref[...] = (p0_ref[...] + p1_ref[...]) * n_ref[...]


_tc_last = pl.pallas_call(
    _tc_last_body,
    grid=(GRID,),
    in_specs=[
        pl.BlockSpec((BLK, DIM), lambda i: (i, 0)),
        pl.BlockSpec((BLK, DIM), lambda i: (i, 0)),
        pl.BlockSpec((BLK, 1), lambda i: (i, 0)),
    ],
    out_specs=pl.BlockSpec((BLK, DIM), lambda i: (i, 0)),
    out_shape=jax.ShapeDtypeStruct((N_NODES, DIM), jnp.float32),
)


def kernel(features, edge_index, W0, W1, W2):
    src = edge_index[0].astype(jnp.int32)
    dst = edge_index[1].astype(jnp.int32)
    n_pad = E_PAD - N_EDGES
    pad_s = jnp.zeros((n_pad,), jnp.int32)
    pad_d = N_NODES + (jnp.arange(n_pad, dtype=jnp.int32) % (P_NODES - N_NODES))
    src_p = jnp.concatenate([src, pad_s]).reshape(NW, CHUNKS, CHUNK)
    dst_p = jnp.concatenate([dst, pad_d]).reshape(NW, CHUNKS, CHUNK)

    degp = _sc_deg(dst_p)
    t1, norm = _tc_first(features, W0, degp[0], degp[1])
    p = _sc_scatter(t1, src_p, dst_p)
    t2 = _tc_mid(p[0], p[1], norm, W1)
    p = _sc_scatter(t2, src_p, dst_p)
    t3 = _tc_mid(p[0], p[1], norm, W2)
    p = _sc_scatter(t3, src_p, dst_p)
    return _tc_last(p[0], p[1], norm)
